# baseline (speedup 1.0000x reference)
import functools

import jax
import jax.numpy as jnp
from jax.experimental import pallas as pl
from jax.experimental.pallas import tpu as pltpu

_BLOCK = 2000

_DN_T = (((1,), (1,)), ((), ()))


def _gc_block_kernel(item_ref, user_ref, r_ref, c_ref, Wu_ref, bu_ref,
                     Wv_ref, bv_ref, Wl_ref, bl_ref, u_out_ref, v_out_ref, *,
                     num_classes, hidden):
    x_item = item_ref[...]
    x_user = user_ref[...]
    rcol = r_ref[...]
    ccol = c_ref[...]
    zu = jax.lax.dot_general(x_item, Wu_ref[...], _DN_T,
                             preferred_element_type=jnp.float32)
    zv = jax.lax.dot_general(x_user, Wv_ref[...], _DN_T,
                             preferred_element_type=jnp.float32)
    H = hidden
    un = jnp.zeros_like(zu[:, 0:H])
    vn = jnp.zeros_like(un)
    for cc in range(num_classes):
        sel = rcol == cc
        un += jnp.where(sel, zu[:, cc * H:(cc + 1) * H] + bu_ref[cc:cc + 1, :], 0.0)
        vn += jnp.where(sel, zv[:, cc * H:(cc + 1) * H] + bv_ref[cc:cc + 1, :], 0.0)
    hu = jnp.maximum(ccol * un, 0.0)
    hv = jnp.maximum(ccol * vn, 0.0)
    ou = jax.lax.dot_general(hu, Wl_ref[...], _DN_T,
                             preferred_element_type=jnp.float32) + bl_ref[...]
    ov = jax.lax.dot_general(hv, Wl_ref[...], _DN_T,
                             preferred_element_type=jnp.float32) + bl_ref[...]
    u_out_ref[...] = jnp.maximum(ou, 0.0)
    v_out_ref[...] = jnp.maximum(ov, 0.0)


def kernel(user, item, r, c, Wu, bu, Wv, bv, Wl, bl):
    N, I = user.shape
    C, H, _ = Wu.shape
    O = Wl.shape[0]
    Wu2 = Wu.reshape(C * H, I)
    Wv2 = Wv.reshape(C * H, I)
    r2 = r.reshape(N, 1).astype(jnp.int32)
    c2 = c.reshape(N, 1)
    nb = N // _BLOCK
    bs_x = pl.BlockSpec((_BLOCK, I), lambda i: (i, 0))
    bs_i = pl.BlockSpec((_BLOCK, 1), lambda i: (i, 0))
    bs_W = pl.BlockSpec((C * H, I), lambda i: (0, 0))
    bs_b = pl.BlockSpec((C, H), lambda i: (0, 0))
    bs_Wl = pl.BlockSpec((O, H), lambda i: (0, 0))
    bs_bl = pl.BlockSpec((1, O), lambda i: (0, 0))
    bs_out = pl.BlockSpec((_BLOCK, O), lambda i: (i, 0))
    u_out, v_out = pl.pallas_call(
        functools.partial(_gc_block_kernel, num_classes=C, hidden=H),
        grid=(nb,),
        in_specs=[bs_x, bs_x, bs_i, bs_i, bs_W, bs_b, bs_W, bs_b, bs_Wl, bs_bl],
        out_specs=[bs_out, bs_out],
        out_shape=[jax.ShapeDtypeStruct((N, O), jnp.float32)] * 2,
        compiler_params=pltpu.CompilerParams(
            dimension_semantics=("parallel",)),
    )(item, user, r2, c2, Wu2, bu, Wv2, bv, Wl, bl.reshape(1, O))
    return (u_out, v_out)
